# baseline (speedup 1.0000x reference)
import functools

import jax
import jax.numpy as jnp
from jax import lax
from jax.experimental import pallas as pl
from jax.experimental.pallas import tpu as pltpu
from jax.experimental.pallas import tpu_sc as plsc

NC = 2
NS = 16
NW = NC * NS
CH1 = 80
CHG1 = 8
CH2 = 128
CHG2 = 8


def _k1_gather_scale(x, meta1, w1, n_nodes, e_pad, n_chunks):
    n_groups = n_chunks // CHG1
    rps = -(-(n_nodes // NS) // 8) * 8
    rps_last = n_nodes - (NS - 1) * rps
    per_w = n_chunks * CH1

    mesh = plsc.VectorSubcoreMesh(core_axis_name="c", subcore_axis_name="s")

    @functools.partial(
        pl.kernel,
        out_type=jax.ShapeDtypeStruct((e_pad, 128), jnp.float32),
        mesh=mesh,
        scratch_types=[
            pltpu.VMEM((2 * CHG1, CH1), jnp.int32),
            pltpu.VMEM((2 * CHG1, CH1), jnp.float32),
            pltpu.VMEM((CH1, 128), jnp.float32),
            pltpu.VMEM((CH1, 128), jnp.float32),
            pltpu.VMEM((CH1, 128), jnp.float32),
            pltpu.VMEM((CH1, 128), jnp.float32),
            pltpu.VMEM_SHARED((n_nodes, 128), jnp.float32),
            pltpu.SemaphoreType.DMA,
            pltpu.SemaphoreType.DMA,
            pltpu.SemaphoreType.DMA,
            pltpu.SemaphoreType.DMA,
            pltpu.SemaphoreType.DMA,
            pltpu.SemaphoreType.DMA,
            pltpu.SemaphoreType.DMA,
            pltpu.SemaphoreType.DMA,
            pltpu.SemaphoreType.DMA,
        ],
    )
    def body(x_hbm, meta_hbm, w_hbm, out_hbm,
             meta_v, w_v, r0, r1, r2, r3, x_sh,
             g0, g1, g2, g3, s0, s1, s2, s3, ms):
        c = lax.axis_index("c")
        s = lax.axis_index("s")
        wid = s * NC + c
        wbase = wid * per_w
        rows = [r0, r1, r2, r3]
        gs = [g0, g1, g2, g3]
        ws = [s0, s1, s2, s3]

        pltpu.sync_copy(meta_hbm.at[wid, 0], meta_v.at[pl.ds(0, CHG1)])
        pltpu.sync_copy(w_hbm.at[wid, 0], w_v.at[pl.ds(0, CHG1)])

        def stage(nrows):
            base = s * rps
            for i in range(nrows // CH1):
                pltpu.sync_copy(x_hbm.at[pl.ds(base + i * CH1, CH1)], r0)
                pltpu.sync_copy(r0, x_sh.at[pl.ds(base + i * CH1, CH1)])
            rem = nrows - (nrows // CH1) * CH1
            if rem:
                off = base + (nrows // CH1) * CH1
                pltpu.sync_copy(x_hbm.at[pl.ds(off, rem)], r0.at[pl.ds(0, rem)])
                pltpu.sync_copy(r0.at[pl.ds(0, rem)], x_sh.at[pl.ds(off, rem)])

        @pl.when(s < NS - 1)
        def _():
            stage(rps)

        @pl.when(s == NS - 1)
        def _():
            stage(rps_last)

        plsc.subcore_barrier()

        pltpu.async_copy(x_sh.at[meta_v.at[0]], rows[0], gs[0])
        pltpu.async_copy(x_sh.at[meta_v.at[1]], rows[1], gs[1])

        def do_scale(rows_v, wrow):
            def scale(kk, _):
                wvec = w_v[wrow, pl.ds(kk * 16, 16)]
                for l in range(16):
                    wk = wvec[l]
                    row = kk * 16 + l
                    for cc in range(8):
                        sl = pl.ds(cc * 16, 16)
                        rows_v[row, sl] = rows_v[row, sl] * wk
                return 0
            lax.fori_loop(0, CH1 // 16, scale, 0)

        def quad(q, _):
            g = q // 2

            @pl.when((q == 2 * g) & (g + 1 < n_groups))
            def _():
                nb = (g + 1) % 2
                pltpu.async_copy(meta_hbm.at[wid, g + 1],
                                 meta_v.at[pl.ds(nb * CHG1, CHG1)], ms)
                pltpu.async_copy(w_hbm.at[wid, g + 1],
                                 w_v.at[pl.ds(nb * CHG1, CHG1)], ms)

            @pl.when((q == 2 * g + 1) & (g + 1 < n_groups))
            def _():
                nb = (g + 1) % 2
                pltpu.make_async_copy(meta_hbm.at[wid, g + 1],
                                      meta_v.at[pl.ds(nb * CHG1, CHG1)], ms).wait()
                pltpu.make_async_copy(w_hbm.at[wid, g + 1],
                                      w_v.at[pl.ds(nb * CHG1, CHG1)], ms).wait()

            for u in range(4):
                j = 4 * q + u
                gj = j // CHG1
                cj = j - gj * CHG1
                mrow = (gj % 2) * CHG1 + cj
                pltpu.make_async_copy(x_sh.at[meta_v.at[mrow]],
                                      rows[u], gs[u]).wait()
                do_scale(rows[u], mrow)
                pltpu.async_copy(rows[u],
                                 out_hbm.at[pl.ds(wbase + j * CH1, CH1)], ws[u])

                v = (u + 2) % 4
                jv = j + 2

                @pl.when(j >= 2)
                def _():
                    pltpu.make_async_copy(
                        rows[v], out_hbm.at[pl.ds(wbase + (j - 2) * CH1, CH1)],
                        ws[v]).wait()

                @pl.when(jv < n_chunks)
                def _():
                    gn = jv // CHG1
                    pltpu.async_copy(
                        x_sh.at[meta_v.at[(gn % 2) * CHG1 + jv - gn * CHG1]],
                        rows[v], gs[v])
            return 0
        lax.fori_loop(0, n_chunks // 4, quad, 0)

        for j in (n_chunks - 2, n_chunks - 1):
            pltpu.make_async_copy(
                rows[j % 4], out_hbm.at[pl.ds(wbase + j * CH1, CH1)],
                ws[j % 4]).wait()

    return body(x, meta1, w1)


def _k2_scatter(msgs, meta2, n_nodes, n_chunks):
    n_groups = n_chunks // CHG2
    ppg = CHG2 // 2
    per_w = n_chunks * CH2
    rps = -(-(n_nodes // NS) // 8) * 8
    rps_last = n_nodes - (NS - 1) * rps

    mesh = plsc.VectorSubcoreMesh(core_axis_name="c", subcore_axis_name="s")

    @functools.partial(
        pl.kernel,
        out_type=jax.ShapeDtypeStruct((NC, n_nodes, 128), jnp.float32),
        mesh=mesh,
        scratch_types=[
            pltpu.VMEM((2 * CHG2, CH2), jnp.int32),
            pltpu.VMEM((CH2, 128), jnp.float32),
            pltpu.VMEM((CH2, 128), jnp.float32),
            pltpu.VMEM_SHARED((n_nodes, 128), jnp.float32),
            pltpu.SemaphoreType.DMA,
            pltpu.SemaphoreType.DMA,
            pltpu.SemaphoreType.DMA,
        ],
    )
    def body(msgs_hbm, meta_hbm, out_hbm,
             meta_v, ma, mb_, acc_sh, ra, rb, ms):
        c = lax.axis_index("c")
        s = lax.axis_index("s")
        wid = s * NC + c
        wbase = wid * per_w

        pltpu.sync_copy(meta_hbm.at[wid, 0], meta_v.at[pl.ds(0, CHG2)])

        def zrow(i, _):
            for cc in range(8):
                ma[i, pl.ds(cc * 16, 16)] = jnp.zeros((16,), jnp.float32)
            return 0
        lax.fori_loop(0, CH2, zrow, 0)

        def zero_slice(nrows):
            base = s * rps
            for i in range(nrows // CH2):
                pltpu.sync_copy(ma, acc_sh.at[pl.ds(base + i * CH2, CH2)])
            rem = nrows - (nrows // CH2) * CH2
            if rem:
                pltpu.sync_copy(ma.at[pl.ds(0, rem)],
                                acc_sh.at[pl.ds(base + (nrows // CH2) * CH2, rem)])

        @pl.when(s < NS - 1)
        def _():
            zero_slice(rps)

        @pl.when(s == NS - 1)
        def _():
            zero_slice(rps_last)

        plsc.subcore_barrier()

        pltpu.async_copy(msgs_hbm.at[pl.ds(wbase, CH2)], ma, ra)
        pltpu.async_copy(msgs_hbm.at[pl.ds(wbase + CH2, CH2)], mb_, rb)

        def pair(p, _):
            g = p // ppg
            mbrow = (g % 2) * CHG2
            cj0 = 2 * (p - g * ppg)
            j0 = 2 * p
            j1 = j0 + 1

            @pl.when((p == g * ppg) & (g + 1 < n_groups))
            def _():
                pltpu.async_copy(meta_hbm.at[wid, g + 1],
                                 meta_v.at[pl.ds(((g + 1) % 2) * CHG2, CHG2)],
                                 ms)

            @pl.when((p == g * ppg + ppg - 1) & (g + 1 < n_groups))
            def _():
                pltpu.make_async_copy(
                    meta_hbm.at[wid, g + 1],
                    meta_v.at[pl.ds(((g + 1) % 2) * CHG2, CHG2)], ms).wait()

            pltpu.make_async_copy(msgs_hbm.at[pl.ds(wbase + j0 * CH2, CH2)],
                                  ma, ra).wait()
            pltpu.sync_copy(ma, acc_sh.at[meta_v.at[mbrow + cj0]], add=True)

            @pl.when(j0 + 2 < n_chunks)
            def _():
                pltpu.async_copy(
                    msgs_hbm.at[pl.ds(wbase + (j0 + 2) * CH2, CH2)], ma, ra)

            pltpu.make_async_copy(msgs_hbm.at[pl.ds(wbase + j1 * CH2, CH2)],
                                  mb_, rb).wait()
            pltpu.sync_copy(mb_, acc_sh.at[meta_v.at[mbrow + cj0 + 1]],
                            add=True)

            @pl.when(j1 + 2 < n_chunks)
            def _():
                pltpu.async_copy(
                    msgs_hbm.at[pl.ds(wbase + (j1 + 2) * CH2, CH2)], mb_, rb)
            return 0
        lax.fori_loop(0, n_chunks // 2, pair, 0)

        plsc.subcore_barrier()

        @pl.when(s < NS - 1)
        def _():
            pltpu.sync_copy(acc_sh.at[pl.ds(s * rps, rps)],
                            out_hbm.at[c, pl.ds(s * rps, rps)])

        @pl.when(s == NS - 1)
        def _():
            pltpu.sync_copy(acc_sh.at[pl.ds((NS - 1) * rps, rps_last)],
                            out_hbm.at[c, pl.ds((NS - 1) * rps, rps_last)])

    return body(msgs, meta2)


def _tc_dense(p, W, b, n_nodes, d_feat, n_units, blk):
    def body(p_ref, w_ref, b_ref, o_ref):
        ah = p_ref[0] + p_ref[1]
        acc = jnp.dot(ah, w_ref[...], preferred_element_type=jnp.float32)
        o_ref[...] = jnp.maximum(acc + b_ref[...], 0.0)

    grid = (n_nodes // blk,)
    return pl.pallas_call(
        body,
        grid=grid,
        in_specs=[
            pl.BlockSpec((2, blk, d_feat), lambda i: (0, i, 0)),
            pl.BlockSpec((d_feat, n_units), lambda i: (0, 0)),
            pl.BlockSpec((1, n_units), lambda i: (0, 0)),
        ],
        out_specs=pl.BlockSpec((blk, n_units), lambda i: (i, 0)),
        out_shape=jax.ShapeDtypeStruct((n_nodes, n_units), jnp.float32),
    )(p, W, b.reshape(1, n_units))


def kernel(x, edge_index, edge_weight, W, b):
    n_nodes, d_feat = x.shape
    n_units = W.shape[1]
    n_edges = edge_weight.shape[0]

    src = edge_index[0].astype(jnp.int32)
    dst = edge_index[1].astype(jnp.int32)
    w = edge_weight.astype(jnp.float32)

    unit = CH1 * CH2 // 16
    per_w = -(-n_edges // (NW * unit)) * unit
    e_pad = NW * per_w
    pad = e_pad - n_edges
    if pad:
        src = jnp.concatenate([src, jnp.zeros((pad,), jnp.int32)])
        dst = jnp.concatenate([dst, jnp.zeros((pad,), jnp.int32)])
        w = jnp.concatenate([w, jnp.zeros((pad,), jnp.float32)])

    n_chunks1 = per_w // CH1
    n_groups1 = n_chunks1 // CHG1
    meta1 = src.reshape(NW, n_groups1, CHG1, CH1)
    w1 = w.reshape(NW, n_groups1, CHG1, CH1)

    n_chunks2 = per_w // CH2
    n_groups2 = n_chunks2 // CHG2
    meta2 = dst.reshape(NW, n_groups2, CHG2, CH2)

    msgs = _k1_gather_scale(x, meta1, w1, n_nodes, e_pad, n_chunks1)
    p = _k2_scatter(msgs, meta2, n_nodes, n_chunks2)
    return _tc_dense(p, W, b, n_nodes, d_feat, n_units, blk=1000)


# speedup vs baseline: 1.0070x; 1.0070x over previous
import functools

import jax
import jax.numpy as jnp
from jax import lax
from jax.experimental import pallas as pl
from jax.experimental.pallas import tpu as pltpu
from jax.experimental.pallas import tpu_sc as plsc

NC = 2
NS = 16
NW = NC * NS
CH1 = 80
CHG1 = 8
CH2 = 128
CHG2 = 8


def _k1_gather_scale(x, meta1, w1, n_nodes, e_pad, n_chunks):
    n_groups = n_chunks // CHG1
    rps = -(-(n_nodes // NS) // 8) * 8
    rps_last = n_nodes - (NS - 1) * rps
    per_w = n_chunks * CH1

    mesh = plsc.VectorSubcoreMesh(core_axis_name="c", subcore_axis_name="s")

    @functools.partial(
        pl.kernel,
        out_type=jax.ShapeDtypeStruct((e_pad, 128), jnp.float32),
        mesh=mesh,
        scratch_types=[
            pltpu.VMEM((2 * CHG1, CH1), jnp.int32),
            pltpu.VMEM((2 * CHG1, CH1), jnp.float32),
            pltpu.VMEM((CH1, 128), jnp.float32),
            pltpu.VMEM((CH1, 128), jnp.float32),
            pltpu.VMEM((CH1, 128), jnp.float32),
            pltpu.VMEM((CH1, 128), jnp.float32),
            pltpu.VMEM_SHARED((n_nodes, 128), jnp.float32),
            pltpu.SemaphoreType.DMA,
            pltpu.SemaphoreType.DMA,
            pltpu.SemaphoreType.DMA,
            pltpu.SemaphoreType.DMA,
            pltpu.SemaphoreType.DMA,
            pltpu.SemaphoreType.DMA,
            pltpu.SemaphoreType.DMA,
            pltpu.SemaphoreType.DMA,
            pltpu.SemaphoreType.DMA,
        ],
    )
    def body(x_hbm, meta_hbm, w_hbm, out_hbm,
             meta_v, w_v, r0, r1, r2, r3, x_sh,
             g0, g1, g2, g3, s0, s1, s2, s3, ms):
        c = lax.axis_index("c")
        s = lax.axis_index("s")
        wid = s * NC + c
        wbase = wid * per_w
        rows = [r0, r1, r2, r3]
        gs = [g0, g1, g2, g3]
        ws = [s0, s1, s2, s3]

        pltpu.sync_copy(meta_hbm.at[wid, 0], meta_v.at[pl.ds(0, CHG1)])
        pltpu.sync_copy(w_hbm.at[wid, 0], w_v.at[pl.ds(0, CHG1)])

        def stage(nrows):
            base = s * rps
            pltpu.sync_copy(x_hbm.at[pl.ds(base, nrows)],
                            x_sh.at[pl.ds(base, nrows)])

        @pl.when(s < NS - 1)
        def _():
            stage(rps)

        @pl.when(s == NS - 1)
        def _():
            stage(rps_last)

        plsc.subcore_barrier()

        pltpu.async_copy(x_sh.at[meta_v.at[0]], rows[0], gs[0])
        pltpu.async_copy(x_sh.at[meta_v.at[1]], rows[1], gs[1])

        def do_scale(rows_v, wrow):
            def scale(kk, _):
                wvec = w_v[wrow, pl.ds(kk * 16, 16)]
                for l in range(16):
                    wk = wvec[l]
                    row = kk * 16 + l
                    for cc in range(8):
                        sl = pl.ds(cc * 16, 16)
                        rows_v[row, sl] = rows_v[row, sl] * wk
                return 0
            lax.fori_loop(0, CH1 // 16, scale, 0)

        def quad(q, _):
            g = q // 2

            @pl.when((q == 2 * g) & (g + 1 < n_groups))
            def _():
                nb = (g + 1) % 2
                pltpu.async_copy(meta_hbm.at[wid, g + 1],
                                 meta_v.at[pl.ds(nb * CHG1, CHG1)], ms)
                pltpu.async_copy(w_hbm.at[wid, g + 1],
                                 w_v.at[pl.ds(nb * CHG1, CHG1)], ms)

            @pl.when((q == 2 * g + 1) & (g + 1 < n_groups))
            def _():
                nb = (g + 1) % 2
                pltpu.make_async_copy(meta_hbm.at[wid, g + 1],
                                      meta_v.at[pl.ds(nb * CHG1, CHG1)], ms).wait()
                pltpu.make_async_copy(w_hbm.at[wid, g + 1],
                                      w_v.at[pl.ds(nb * CHG1, CHG1)], ms).wait()

            for u in range(4):
                j = 4 * q + u
                gj = j // CHG1
                cj = j - gj * CHG1
                mrow = (gj % 2) * CHG1 + cj
                pltpu.make_async_copy(x_sh.at[meta_v.at[mrow]],
                                      rows[u], gs[u]).wait()
                do_scale(rows[u], mrow)
                pltpu.async_copy(rows[u],
                                 out_hbm.at[pl.ds(wbase + j * CH1, CH1)], ws[u])

                v = (u + 2) % 4
                jv = j + 2

                @pl.when(j >= 2)
                def _():
                    pltpu.make_async_copy(
                        rows[v], out_hbm.at[pl.ds(wbase + (j - 2) * CH1, CH1)],
                        ws[v]).wait()

                @pl.when(jv < n_chunks)
                def _():
                    gn = jv // CHG1
                    pltpu.async_copy(
                        x_sh.at[meta_v.at[(gn % 2) * CHG1 + jv - gn * CHG1]],
                        rows[v], gs[v])
            return 0
        lax.fori_loop(0, n_chunks // 4, quad, 0)

        for j in (n_chunks - 2, n_chunks - 1):
            pltpu.make_async_copy(
                rows[j % 4], out_hbm.at[pl.ds(wbase + j * CH1, CH1)],
                ws[j % 4]).wait()

    return body(x, meta1, w1)


def _k2_scatter(msgs, meta2, n_nodes, n_chunks):
    n_groups = n_chunks // CHG2
    ppg = CHG2 // 2
    per_w = n_chunks * CH2
    rps = -(-(n_nodes // NS) // 8) * 8
    rps_last = n_nodes - (NS - 1) * rps

    mesh = plsc.VectorSubcoreMesh(core_axis_name="c", subcore_axis_name="s")

    @functools.partial(
        pl.kernel,
        out_type=jax.ShapeDtypeStruct((NC, n_nodes, 128), jnp.float32),
        mesh=mesh,
        scratch_types=[
            pltpu.VMEM((2 * CHG2, CH2), jnp.int32),
            pltpu.VMEM((CH2, 128), jnp.float32),
            pltpu.VMEM((CH2, 128), jnp.float32),
            pltpu.VMEM_SHARED((n_nodes, 128), jnp.float32),
            pltpu.SemaphoreType.DMA,
            pltpu.SemaphoreType.DMA,
            pltpu.SemaphoreType.DMA,
        ],
    )
    def body(msgs_hbm, meta_hbm, out_hbm,
             meta_v, ma, mb_, acc_sh, ra, rb, ms):
        c = lax.axis_index("c")
        s = lax.axis_index("s")
        wid = s * NC + c
        wbase = wid * per_w

        pltpu.sync_copy(meta_hbm.at[wid, 0], meta_v.at[pl.ds(0, CHG2)])

        def zrow(i, _):
            for cc in range(8):
                ma[i, pl.ds(cc * 16, 16)] = jnp.zeros((16,), jnp.float32)
            return 0
        lax.fori_loop(0, CH2, zrow, 0)

        def zero_slice(nrows):
            base = s * rps
            for i in range(nrows // CH2):
                pltpu.sync_copy(ma, acc_sh.at[pl.ds(base + i * CH2, CH2)])
            rem = nrows - (nrows // CH2) * CH2
            if rem:
                pltpu.sync_copy(ma.at[pl.ds(0, rem)],
                                acc_sh.at[pl.ds(base + (nrows // CH2) * CH2, rem)])

        @pl.when(s < NS - 1)
        def _():
            zero_slice(rps)

        @pl.when(s == NS - 1)
        def _():
            zero_slice(rps_last)

        plsc.subcore_barrier()

        pltpu.async_copy(msgs_hbm.at[pl.ds(wbase, CH2)], ma, ra)
        pltpu.async_copy(msgs_hbm.at[pl.ds(wbase + CH2, CH2)], mb_, rb)

        def pair(p, _):
            g = p // ppg
            mbrow = (g % 2) * CHG2
            cj0 = 2 * (p - g * ppg)
            j0 = 2 * p
            j1 = j0 + 1

            @pl.when((p == g * ppg) & (g + 1 < n_groups))
            def _():
                pltpu.async_copy(meta_hbm.at[wid, g + 1],
                                 meta_v.at[pl.ds(((g + 1) % 2) * CHG2, CHG2)],
                                 ms)

            @pl.when((p == g * ppg + ppg - 1) & (g + 1 < n_groups))
            def _():
                pltpu.make_async_copy(
                    meta_hbm.at[wid, g + 1],
                    meta_v.at[pl.ds(((g + 1) % 2) * CHG2, CHG2)], ms).wait()

            pltpu.make_async_copy(msgs_hbm.at[pl.ds(wbase + j0 * CH2, CH2)],
                                  ma, ra).wait()
            pltpu.sync_copy(ma, acc_sh.at[meta_v.at[mbrow + cj0]], add=True)

            @pl.when(j0 + 2 < n_chunks)
            def _():
                pltpu.async_copy(
                    msgs_hbm.at[pl.ds(wbase + (j0 + 2) * CH2, CH2)], ma, ra)

            pltpu.make_async_copy(msgs_hbm.at[pl.ds(wbase + j1 * CH2, CH2)],
                                  mb_, rb).wait()
            pltpu.sync_copy(mb_, acc_sh.at[meta_v.at[mbrow + cj0 + 1]],
                            add=True)

            @pl.when(j1 + 2 < n_chunks)
            def _():
                pltpu.async_copy(
                    msgs_hbm.at[pl.ds(wbase + (j1 + 2) * CH2, CH2)], mb_, rb)
            return 0
        lax.fori_loop(0, n_chunks // 2, pair, 0)

        plsc.subcore_barrier()

        @pl.when(s < NS - 1)
        def _():
            pltpu.sync_copy(acc_sh.at[pl.ds(s * rps, rps)],
                            out_hbm.at[c, pl.ds(s * rps, rps)])

        @pl.when(s == NS - 1)
        def _():
            pltpu.sync_copy(acc_sh.at[pl.ds((NS - 1) * rps, rps_last)],
                            out_hbm.at[c, pl.ds((NS - 1) * rps, rps_last)])

    return body(msgs, meta2)


def _tc_dense(p, W, b, n_nodes, d_feat, n_units, blk):
    def body(p_ref, w_ref, b_ref, o_ref):
        ah = p_ref[0] + p_ref[1]
        acc = jnp.dot(ah, w_ref[...], preferred_element_type=jnp.float32)
        o_ref[...] = jnp.maximum(acc + b_ref[...], 0.0)

    grid = (n_nodes // blk,)
    return pl.pallas_call(
        body,
        grid=grid,
        in_specs=[
            pl.BlockSpec((2, blk, d_feat), lambda i: (0, i, 0)),
            pl.BlockSpec((d_feat, n_units), lambda i: (0, 0)),
            pl.BlockSpec((1, n_units), lambda i: (0, 0)),
        ],
        out_specs=pl.BlockSpec((blk, n_units), lambda i: (i, 0)),
        out_shape=jax.ShapeDtypeStruct((n_nodes, n_units), jnp.float32),
    )(p, W, b.reshape(1, n_units))


def kernel(x, edge_index, edge_weight, W, b):
    n_nodes, d_feat = x.shape
    n_units = W.shape[1]
    n_edges = edge_weight.shape[0]

    src = edge_index[0].astype(jnp.int32)
    dst = edge_index[1].astype(jnp.int32)
    w = edge_weight.astype(jnp.float32)

    unit = CH1 * CH2 // 16
    per_w = -(-n_edges // (NW * unit)) * unit
    e_pad = NW * per_w
    pad = e_pad - n_edges
    if pad:
        src = jnp.concatenate([src, jnp.zeros((pad,), jnp.int32)])
        dst = jnp.concatenate([dst, jnp.zeros((pad,), jnp.int32)])
        w = jnp.concatenate([w, jnp.zeros((pad,), jnp.float32)])

    n_chunks1 = per_w // CH1
    n_groups1 = n_chunks1 // CHG1
    meta1 = src.reshape(NW, n_groups1, CHG1, CH1)
    w1 = w.reshape(NW, n_groups1, CHG1, CH1)

    n_chunks2 = per_w // CH2
    n_groups2 = n_chunks2 // CHG2
    meta2 = dst.reshape(NW, n_groups2, CHG2, CH2)

    msgs = _k1_gather_scale(x, meta1, w1, n_nodes, e_pad, n_chunks1)
    p = _k2_scatter(msgs, meta2, n_nodes, n_chunks2)
    return _tc_dense(p, W, b, n_nodes, d_feat, n_units, blk=1000)
